# baseline (speedup 1.0000x reference)
import functools

import jax
import jax.numpy as jnp
from jax import lax
from jax.experimental import pallas as pl
from jax.experimental.pallas import tpu as pltpu
from jax.experimental.pallas import tpu_sc as plsc

_N_ATOMS = 6400000
_N_MOL = 100000
_NC = 2
_NS = 16
_NW = _NC * _NS
_CHUNK = _N_ATOMS // _NW
_BLK = 4000
_NBLK = _CHUNK // _BLK
_U = 10
_PAD = 102400
_SL_SC = _PAD // _NS
_SL_CB = _PAD // _NW
_FC = 128


@functools.lru_cache(maxsize=None)
def _build():
    mesh = plsc.VectorSubcoreMesh(core_axis_name="c", subcore_axis_name="s",
                                  num_cores=_NC, num_subcores=_NS)

    @functools.partial(
        pl.kernel,
        out_type=jax.ShapeDtypeStruct((_NC * _PAD,), jnp.float32),
        mesh=mesh,
        compiler_params=pltpu.CompilerParams(needs_layout_passes=False),
        scratch_types=[
            pltpu.VMEM((_BLK + 16,), jnp.int32),
            pltpu.VMEM((_BLK + 16,), jnp.int32),
            pltpu.VMEM((_BLK,), jnp.float32),
            pltpu.VMEM((_BLK,), jnp.float32),
            pltpu.VMEM((_PAD,), jnp.float32),
            pltpu.VMEM((1, _FC), jnp.int32),
            pltpu.VMEM_SHARED((_PAD,), jnp.float32),
            pltpu.SemaphoreType.DMA,
            pltpu.SemaphoreType.DMA,
            pltpu.SemaphoreType.DMA,
            pltpu.SemaphoreType.DMA,
        ],
    )
    def segsum_partials(idx_hbm, val_hbm, out_hbm, idx_b0, idx_b1, val_b0,
                        val_b1, acc_v, ramp_v, acc_sh, sem_i0, sem_i1,
                        sem_v0, sem_v1):
        c = lax.axis_index("c")
        s = lax.axis_index("s")
        wid = s * _NC + c
        a0 = wid * _CHUNK
        zeros16 = jnp.zeros((16,), jnp.float32)

        pltpu.async_copy(idx_hbm.at[pl.ds(a0, _BLK)],
                         idx_b0.at[pl.ds(0, _BLK)], sem_i0)
        pltpu.async_copy(val_hbm.at[pl.ds(a0, _BLK)], val_b0, sem_v0)

        def zero_body(i, carry):
            acc_v[pl.ds(i * 16, 16)] = zeros16
            return carry

        lax.fori_loop(0, _PAD // 16, zero_body, 0)

        pltpu.sync_copy(acc_v.at[pl.ds(0, _SL_SC)],
                        acc_sh.at[pl.ds(s * _SL_SC, _SL_SC)])
        plsc.subcore_barrier()

        pltpu.sync_copy(idx_hbm.at[pl.ds(a0, _FC)], ramp_v.at[0])
        lo = ramp_v[0, pl.ds(0, 16)][0]

        bufs = ((idx_b0, val_b0, sem_i0, sem_v0),
                (idx_b1, val_b1, sem_i1, sem_v1))

        def pair(gp, carry):
            for p in range(2):
                ib, vb, si, sv = bufs[p]
                nib, nvb, nsi, nsv = bufs[1 - p]
                n = 2 * gp + p
                base = a0 + n * _BLK
                pltpu.make_async_copy(idx_hbm.at[pl.ds(base, _BLK)],
                                      ib.at[pl.ds(0, _BLK)], si).wait()
                pltpu.make_async_copy(val_hbm.at[pl.ds(base, _BLK)],
                                      vb, sv).wait()

                @pl.when(n + 1 < _NBLK)
                def _prefetch():
                    nbase = a0 + (n + 1) * _BLK
                    pltpu.async_copy(idx_hbm.at[pl.ds(nbase, _BLK)],
                                     nib.at[pl.ds(0, _BLK)], nsi)
                    pltpu.async_copy(val_hbm.at[pl.ds(nbase, _BLK)],
                                     nvb, nsv)

                def grp(g, inner):
                    gb = g * (16 * _U)
                    for u in range(_U):
                        o = gb + u * 16
                        i16 = ib[pl.ds(o, 16)]
                        v16 = vb[pl.ds(o, 16)]
                        plsc.addupdate_scatter(acc_v, [i16], v16)
                    return inner

                lax.fori_loop(0, _BLK // (16 * _U), grp, 0)
            return carry

        lax.fori_loop(0, _NBLK // 2, pair, 0)

        hi = idx_b1[pl.ds(_BLK - 16, 16)][15]

        iota16 = lax.iota(jnp.int32, 16)

        def flush(j, carry):
            for k in range(_FC // 16):
                ramp_v[0, pl.ds(k * 16, 16)] = j * _FC + k * 16 + iota16
            pltpu.sync_copy(acc_v.at[pl.ds(j * _FC, _FC)],
                            acc_sh.at[ramp_v.at[0]], add=True)
            return carry

        lax.fori_loop(lo // _FC, hi // _FC + 1, flush, 0)
        plsc.subcore_barrier()

        pltpu.sync_copy(acc_sh.at[pl.ds(s * _SL_SC, _SL_SC)],
                        out_hbm.at[pl.ds(c * _PAD + s * _SL_SC, _SL_SC)])

    @functools.partial(
        pl.kernel,
        out_type=jax.ShapeDtypeStruct((_PAD,), jnp.float32),
        mesh=mesh,
        compiler_params=pltpu.CompilerParams(needs_layout_passes=False),
        scratch_types=[
            pltpu.VMEM((_SL_CB,), jnp.float32),
            pltpu.VMEM((_SL_CB,), jnp.float32),
        ],
    )
    def combine(part_hbm, out_hbm, a_v, b_v):
        c = lax.axis_index("c")
        s = lax.axis_index("s")
        wid = s * _NC + c
        base = wid * _SL_CB
        pltpu.sync_copy(part_hbm.at[pl.ds(base, _SL_CB)], a_v)
        pltpu.sync_copy(part_hbm.at[pl.ds(_PAD + base, _SL_CB)], b_v)

        def body(i, carry):
            a_v[pl.ds(i * 16, 16)] = (a_v[pl.ds(i * 16, 16)] +
                                      b_v[pl.ds(i * 16, 16)])
            return carry

        lax.fori_loop(0, _SL_CB // 16, body, 0)
        pltpu.sync_copy(a_v, out_hbm.at[pl.ds(base, _SL_CB)])

    return segsum_partials, combine


def kernel(indices, per_atom_property):
    segsum_partials, combine = _build()
    idx = indices.astype(jnp.int32)
    val = per_atom_property.reshape(-1).astype(jnp.float32)
    partials = segsum_partials(idx, val)
    full = combine(partials)
    return full[:_N_MOL].reshape(_N_MOL, 1)


# speedup vs baseline: 1.5416x; 1.5416x over previous
import functools

import jax
import jax.numpy as jnp
from jax import lax
from jax.experimental import pallas as pl
from jax.experimental.pallas import tpu as pltpu
from jax.experimental.pallas import tpu_sc as plsc

_N_ATOMS = 6400000
_N_MOL = 100000
_NC = 2
_NS = 16
_NW = _NC * _NS
_CHUNK = _N_ATOMS // _NW
_BLK = 4000
_NBLK = _CHUNK // _BLK
_U = 10
_PAD = 102400
_SL_SC = _PAD // _NS
_SL_CB = _PAD // _NW
_FC = 128


@functools.lru_cache(maxsize=None)
def _build():
    mesh = plsc.VectorSubcoreMesh(core_axis_name="c", subcore_axis_name="s",
                                  num_cores=_NC, num_subcores=_NS)

    @functools.partial(
        pl.kernel,
        out_type=jax.ShapeDtypeStruct((_NC * _PAD,), jnp.float32),
        mesh=mesh,
        compiler_params=pltpu.CompilerParams(needs_layout_passes=False),
        scratch_types=[
            pltpu.VMEM((_BLK + 16,), jnp.int32),
            pltpu.VMEM((_BLK + 16,), jnp.int32),
            pltpu.VMEM((_BLK,), jnp.float32),
            pltpu.VMEM((_BLK,), jnp.float32),
            pltpu.VMEM((_PAD,), jnp.float32),
            pltpu.VMEM((1, _FC), jnp.int32),
            pltpu.VMEM_SHARED((_PAD,), jnp.float32),
            pltpu.SemaphoreType.DMA,
            pltpu.SemaphoreType.DMA,
            pltpu.SemaphoreType.DMA,
            pltpu.SemaphoreType.DMA,
        ],
    )
    def segsum_partials(idx_hbm, val_hbm, out_hbm, idx_b0, idx_b1, val_b0,
                        val_b1, acc_v, ramp_v, acc_sh, sem_i0, sem_i1,
                        sem_v0, sem_v1):
        c = lax.axis_index("c")
        s = lax.axis_index("s")
        wid = s * _NC + c
        a0 = wid * _CHUNK
        zeros16 = jnp.zeros((16,), jnp.float32)

        pltpu.async_copy(idx_hbm.at[pl.ds(a0, _BLK)],
                         idx_b0.at[pl.ds(0, _BLK)], sem_i0)
        pltpu.async_copy(val_hbm.at[pl.ds(a0, _BLK)], val_b0, sem_v0)

        def zero_body(i, carry):
            acc_v[pl.ds(i * 16, 16)] = zeros16
            return carry

        lax.fori_loop(0, _PAD // 16, zero_body, 0)

        pltpu.sync_copy(acc_v.at[pl.ds(0, _SL_SC)],
                        acc_sh.at[pl.ds(s * _SL_SC, _SL_SC)])
        plsc.subcore_barrier()

        pltpu.sync_copy(idx_hbm.at[pl.ds(a0, _FC)], ramp_v.at[0])
        lo = ramp_v[0, pl.ds(0, 16)][0]

        iota16 = lax.iota(jnp.int32, 16)
        is15 = iota16 == 15
        not15 = iota16 < 15

        bufs = ((idx_b0, val_b0, sem_i0, sem_v0),
                (idx_b1, val_b1, sem_i1, sem_v1))

        def pair(gp, carry):
            for p in range(2):
                ib, vb, si, sv = bufs[p]
                nib, nvb, nsi, nsv = bufs[1 - p]
                n = 2 * gp + p
                base = a0 + n * _BLK
                pltpu.make_async_copy(idx_hbm.at[pl.ds(base, _BLK)],
                                      ib.at[pl.ds(0, _BLK)], si).wait()
                pltpu.make_async_copy(val_hbm.at[pl.ds(base, _BLK)],
                                      vb, sv).wait()

                @pl.when(n + 1 < _NBLK)
                def _prefetch():
                    nbase = a0 + (n + 1) * _BLK
                    pltpu.async_copy(idx_hbm.at[pl.ds(nbase, _BLK)],
                                     nib.at[pl.ds(0, _BLK)], nsi)
                    pltpu.async_copy(val_hbm.at[pl.ds(nbase, _BLK)],
                                     nvb, nsv)

                def grp(g, inner):
                    gb = g * (16 * _U)
                    for u in range(_U):
                        o = gb + u * 16
                        i16 = ib[pl.ds(o, 16)]
                        nxt = ib[pl.ds(o + 1, 16)]
                        v16 = vb[pl.ds(o, 16)]
                        c = plsc.cumsum(v16)
                        neq = i16 != nxt
                        plsc.addupdate_scatter(acc_v, [i16], c,
                                               mask=neq | is15)
                        plsc.addupdate_scatter(acc_v, [nxt], -c,
                                               mask=neq & not15)
                    return inner

                lax.fori_loop(0, _BLK // (16 * _U), grp, 0)
            return carry

        lax.fori_loop(0, _NBLK // 2, pair, 0)

        hi = idx_b1[pl.ds(_BLK - 16, 16)][15]

        iota16 = lax.iota(jnp.int32, 16)

        def flush(j, carry):
            for k in range(_FC // 16):
                ramp_v[0, pl.ds(k * 16, 16)] = j * _FC + k * 16 + iota16
            pltpu.sync_copy(acc_v.at[pl.ds(j * _FC, _FC)],
                            acc_sh.at[ramp_v.at[0]], add=True)
            return carry

        lax.fori_loop(lo // _FC, hi // _FC + 1, flush, 0)
        plsc.subcore_barrier()

        pltpu.sync_copy(acc_sh.at[pl.ds(s * _SL_SC, _SL_SC)],
                        out_hbm.at[pl.ds(c * _PAD + s * _SL_SC, _SL_SC)])

    @functools.partial(
        pl.kernel,
        out_type=jax.ShapeDtypeStruct((_PAD,), jnp.float32),
        mesh=mesh,
        compiler_params=pltpu.CompilerParams(needs_layout_passes=False),
        scratch_types=[
            pltpu.VMEM((_SL_CB,), jnp.float32),
            pltpu.VMEM((_SL_CB,), jnp.float32),
        ],
    )
    def combine(part_hbm, out_hbm, a_v, b_v):
        c = lax.axis_index("c")
        s = lax.axis_index("s")
        wid = s * _NC + c
        base = wid * _SL_CB
        pltpu.sync_copy(part_hbm.at[pl.ds(base, _SL_CB)], a_v)
        pltpu.sync_copy(part_hbm.at[pl.ds(_PAD + base, _SL_CB)], b_v)

        def body(i, carry):
            a_v[pl.ds(i * 16, 16)] = (a_v[pl.ds(i * 16, 16)] +
                                      b_v[pl.ds(i * 16, 16)])
            return carry

        lax.fori_loop(0, _SL_CB // 16, body, 0)
        pltpu.sync_copy(a_v, out_hbm.at[pl.ds(base, _SL_CB)])

    return segsum_partials, combine


def kernel(indices, per_atom_property):
    segsum_partials, combine = _build()
    idx = indices.astype(jnp.int32)
    val = per_atom_property.reshape(-1).astype(jnp.float32)
    partials = segsum_partials(idx, val)
    full = combine(partials)
    return full[:_N_MOL].reshape(_N_MOL, 1)


# speedup vs baseline: 2.9346x; 1.9036x over previous
import functools

import jax
import jax.numpy as jnp
from jax import lax
from jax.experimental import pallas as pl
from jax.experimental.pallas import tpu as pltpu
from jax.experimental.pallas import tpu_sc as plsc

_N_ATOMS = 6400000
_N_MOL = 100000
_NC = 2
_NS = 16
_NW = _NC * _NS
_CHUNK = _N_ATOMS // _NW
_BLK = 4000
_NBLK = _CHUNK // _BLK
_U = 10
_PAD = 102400
_SL_SC = _PAD // _NS
_SL_CB = _PAD // _NW
_FC = 128


@functools.lru_cache(maxsize=None)
def _build():
    mesh = plsc.VectorSubcoreMesh(core_axis_name="c", subcore_axis_name="s",
                                  num_cores=_NC, num_subcores=_NS)

    @functools.partial(
        pl.kernel,
        out_type=jax.ShapeDtypeStruct((_NC * _PAD,), jnp.float32),
        mesh=mesh,
        compiler_params=pltpu.CompilerParams(needs_layout_passes=False),
        scratch_types=[
            pltpu.VMEM((_BLK + 16,), jnp.int32),
            pltpu.VMEM((_BLK + 16,), jnp.int32),
            pltpu.VMEM((_BLK,), jnp.float32),
            pltpu.VMEM((_BLK,), jnp.float32),
            pltpu.VMEM((_PAD,), jnp.float32),
            pltpu.VMEM((1, _FC), jnp.int32),
            pltpu.VMEM_SHARED((_PAD,), jnp.float32),
            pltpu.SemaphoreType.DMA,
            pltpu.SemaphoreType.DMA,
            pltpu.SemaphoreType.DMA,
            pltpu.SemaphoreType.DMA,
        ],
    )
    def segsum_partials(idx_hbm, val_hbm, out_hbm, idx_b0, idx_b1, val_b0,
                        val_b1, acc_v, ramp_v, acc_sh, sem_i0, sem_i1,
                        sem_v0, sem_v1):
        c = lax.axis_index("c")
        s = lax.axis_index("s")
        wid = s * _NC + c
        a0 = wid * _CHUNK
        zeros16 = jnp.zeros((16,), jnp.float32)

        pltpu.async_copy(idx_hbm.at[pl.ds(a0, _BLK)],
                         idx_b0.at[pl.ds(0, _BLK)], sem_i0)
        pltpu.async_copy(val_hbm.at[pl.ds(a0, _BLK)], val_b0, sem_v0)

        def zero_body(i, carry):
            acc_v[pl.ds(i * 16, 16)] = zeros16
            return carry

        lax.fori_loop(0, _PAD // 16, zero_body, 0)

        pltpu.sync_copy(acc_v.at[pl.ds(0, _SL_SC)],
                        acc_sh.at[pl.ds(s * _SL_SC, _SL_SC)])
        plsc.subcore_barrier()

        pltpu.sync_copy(idx_hbm.at[pl.ds(a0, _FC)], ramp_v.at[0])
        lo = ramp_v[0, pl.ds(0, 16)][0]

        iota16 = lax.iota(jnp.int32, 16)
        is15 = iota16 == 15
        not15 = iota16 < 15

        bufs = ((idx_b0, val_b0, sem_i0, sem_v0),
                (idx_b1, val_b1, sem_i1, sem_v1))

        def pair(gp, carry):
            for p in range(2):
                ib, vb, si, sv = bufs[p]
                nib, nvb, nsi, nsv = bufs[1 - p]
                n = 2 * gp + p
                base = a0 + n * _BLK
                pltpu.make_async_copy(idx_hbm.at[pl.ds(base, _BLK)],
                                      ib.at[pl.ds(0, _BLK)], si).wait()
                pltpu.make_async_copy(val_hbm.at[pl.ds(base, _BLK)],
                                      vb, sv).wait()

                @pl.when(n + 1 < _NBLK)
                def _prefetch():
                    nbase = a0 + (n + 1) * _BLK
                    pltpu.async_copy(idx_hbm.at[pl.ds(nbase, _BLK)],
                                     nib.at[pl.ds(0, _BLK)], nsi)
                    pltpu.async_copy(val_hbm.at[pl.ds(nbase, _BLK)],
                                     nvb, nsv)

                @plsc.parallel_loop(0, _BLK // 16, unroll=_U)
                def _vec(g):
                    o = g * 16
                    i16 = ib[pl.ds(o, 16)]
                    nxt = ib[pl.ds(o + 1, 16)]
                    v16 = vb[pl.ds(o, 16)]
                    c = plsc.cumsum(v16)
                    neq = i16 != nxt
                    plsc.addupdate_scatter(acc_v, [i16], c, mask=neq | is15)
                    plsc.addupdate_scatter(acc_v, [nxt], -c,
                                           mask=neq & not15)
            return carry

        lax.fori_loop(0, _NBLK // 2, pair, 0)

        hi = idx_b1[pl.ds(_BLK - 16, 16)][15]

        iota16 = lax.iota(jnp.int32, 16)

        def flush(j, carry):
            for k in range(_FC // 16):
                ramp_v[0, pl.ds(k * 16, 16)] = j * _FC + k * 16 + iota16
            pltpu.sync_copy(acc_v.at[pl.ds(j * _FC, _FC)],
                            acc_sh.at[ramp_v.at[0]], add=True)
            return carry

        lax.fori_loop(lo // _FC, hi // _FC + 1, flush, 0)
        plsc.subcore_barrier()

        pltpu.sync_copy(acc_sh.at[pl.ds(s * _SL_SC, _SL_SC)],
                        out_hbm.at[pl.ds(c * _PAD + s * _SL_SC, _SL_SC)])

    @functools.partial(
        pl.kernel,
        out_type=jax.ShapeDtypeStruct((_PAD,), jnp.float32),
        mesh=mesh,
        compiler_params=pltpu.CompilerParams(needs_layout_passes=False),
        scratch_types=[
            pltpu.VMEM((_SL_CB,), jnp.float32),
            pltpu.VMEM((_SL_CB,), jnp.float32),
        ],
    )
    def combine(part_hbm, out_hbm, a_v, b_v):
        c = lax.axis_index("c")
        s = lax.axis_index("s")
        wid = s * _NC + c
        base = wid * _SL_CB
        pltpu.sync_copy(part_hbm.at[pl.ds(base, _SL_CB)], a_v)
        pltpu.sync_copy(part_hbm.at[pl.ds(_PAD + base, _SL_CB)], b_v)

        def body(i, carry):
            a_v[pl.ds(i * 16, 16)] = (a_v[pl.ds(i * 16, 16)] +
                                      b_v[pl.ds(i * 16, 16)])
            return carry

        lax.fori_loop(0, _SL_CB // 16, body, 0)
        pltpu.sync_copy(a_v, out_hbm.at[pl.ds(base, _SL_CB)])

    return segsum_partials, combine


def kernel(indices, per_atom_property):
    segsum_partials, combine = _build()
    idx = indices.astype(jnp.int32)
    val = per_atom_property.reshape(-1).astype(jnp.float32)
    partials = segsum_partials(idx, val)
    full = combine(partials)
    return full[:_N_MOL].reshape(_N_MOL, 1)


# speedup vs baseline: 3.9135x; 1.3336x over previous
import functools

import jax
import jax.numpy as jnp
from jax import lax
from jax.experimental import pallas as pl
from jax.experimental.pallas import tpu as pltpu
from jax.experimental.pallas import tpu_sc as plsc

_N_ATOMS = 6400000
_N_MOL = 100000
_NC = 2
_NS = 16
_NW = _NC * _NS
_CHUNK = _N_ATOMS // _NW
_BLK = 4000
_NBLK = _CHUNK // _BLK
_U = 10
_PAD = 102400
_SL_SC = _PAD // _NS
_SL_CB = _PAD // _NW
_FC = 128


@functools.lru_cache(maxsize=None)
def _build():
    mesh = plsc.VectorSubcoreMesh(core_axis_name="c", subcore_axis_name="s",
                                  num_cores=_NC, num_subcores=_NS)

    @functools.partial(
        pl.kernel,
        out_type=jax.ShapeDtypeStruct((_NC * _PAD,), jnp.float32),
        mesh=mesh,
        compiler_params=pltpu.CompilerParams(needs_layout_passes=False),
        scratch_types=[
            pltpu.VMEM((_BLK + 16,), jnp.int32),
            pltpu.VMEM((_BLK + 16,), jnp.int32),
            pltpu.VMEM((_BLK,), jnp.float32),
            pltpu.VMEM((_BLK,), jnp.float32),
            pltpu.VMEM((_PAD,), jnp.float32),
            pltpu.VMEM((1, _FC), jnp.int32),
            pltpu.VMEM_SHARED((_PAD,), jnp.float32),
            pltpu.SemaphoreType.DMA,
            pltpu.SemaphoreType.DMA,
            pltpu.SemaphoreType.DMA,
            pltpu.SemaphoreType.DMA,
        ],
    )
    def segsum_partials(idx_hbm, val_hbm, out_hbm, idx_b0, idx_b1, val_b0,
                        val_b1, acc_v, ramp_v, acc_sh, sem_i0, sem_i1,
                        sem_v0, sem_v1):
        c = lax.axis_index("c")
        s = lax.axis_index("s")
        wid = s * _NC + c
        a0 = wid * _CHUNK
        zeros16 = jnp.zeros((16,), jnp.float32)

        pltpu.async_copy(idx_hbm.at[pl.ds(a0, _BLK)],
                         idx_b0.at[pl.ds(0, _BLK)], sem_i0)
        pltpu.async_copy(val_hbm.at[pl.ds(a0, _BLK)], val_b0, sem_v0)

        @plsc.parallel_loop(0, _PAD // 16, unroll=8)
        def _zero(i):
            acc_v[pl.ds(i * 16, 16)] = zeros16

        pltpu.sync_copy(acc_v.at[pl.ds(0, _SL_SC)],
                        acc_sh.at[pl.ds(s * _SL_SC, _SL_SC)])
        plsc.subcore_barrier()

        pltpu.sync_copy(idx_hbm.at[pl.ds(a0, _FC)], ramp_v.at[0])
        lo = ramp_v[0, pl.ds(0, 16)][0]

        iota16 = lax.iota(jnp.int32, 16)
        is15 = iota16 == 15
        not15 = iota16 < 15

        bufs = ((idx_b0, val_b0, sem_i0, sem_v0),
                (idx_b1, val_b1, sem_i1, sem_v1))

        def pair(gp, carry):
            for p in range(2):
                ib, vb, si, sv = bufs[p]
                nib, nvb, nsi, nsv = bufs[1 - p]
                n = 2 * gp + p
                base = a0 + n * _BLK
                pltpu.make_async_copy(idx_hbm.at[pl.ds(base, _BLK)],
                                      ib.at[pl.ds(0, _BLK)], si).wait()
                pltpu.make_async_copy(val_hbm.at[pl.ds(base, _BLK)],
                                      vb, sv).wait()

                @pl.when(n + 1 < _NBLK)
                def _prefetch():
                    nbase = a0 + (n + 1) * _BLK
                    pltpu.async_copy(idx_hbm.at[pl.ds(nbase, _BLK)],
                                     nib.at[pl.ds(0, _BLK)], nsi)
                    pltpu.async_copy(val_hbm.at[pl.ds(nbase, _BLK)],
                                     nvb, nsv)

                @plsc.parallel_loop(0, _BLK // 16, unroll=_U)
                def _vec(g):
                    o = g * 16
                    i16 = ib[pl.ds(o, 16)]
                    nxt = ib[pl.ds(o + 1, 16)]
                    v16 = vb[pl.ds(o, 16)]
                    c = plsc.cumsum(v16)
                    neq = i16 != nxt
                    plsc.addupdate_scatter(acc_v, [i16], c, mask=neq | is15)
                    plsc.addupdate_scatter(acc_v, [nxt], -c,
                                           mask=neq & not15)
            return carry

        lax.fori_loop(0, _NBLK // 2, pair, 0)

        hi = idx_b1[pl.ds(_BLK - 16, 16)][15]

        iota16 = lax.iota(jnp.int32, 16)

        def flush(j, carry):
            for k in range(_FC // 16):
                ramp_v[0, pl.ds(k * 16, 16)] = j * _FC + k * 16 + iota16
            pltpu.sync_copy(acc_v.at[pl.ds(j * _FC, _FC)],
                            acc_sh.at[ramp_v.at[0]], add=True)
            return carry

        lax.fori_loop(lo // _FC, hi // _FC + 1, flush, 0)
        plsc.subcore_barrier()

        pltpu.sync_copy(acc_sh.at[pl.ds(s * _SL_SC, _SL_SC)],
                        out_hbm.at[pl.ds(c * _PAD + s * _SL_SC, _SL_SC)])

    @functools.partial(
        pl.kernel,
        out_type=jax.ShapeDtypeStruct((_PAD,), jnp.float32),
        mesh=mesh,
        compiler_params=pltpu.CompilerParams(needs_layout_passes=False),
        scratch_types=[
            pltpu.VMEM((_SL_CB,), jnp.float32),
            pltpu.VMEM((_SL_CB,), jnp.float32),
        ],
    )
    def combine(part_hbm, out_hbm, a_v, b_v):
        c = lax.axis_index("c")
        s = lax.axis_index("s")
        wid = s * _NC + c
        base = wid * _SL_CB
        pltpu.sync_copy(part_hbm.at[pl.ds(base, _SL_CB)], a_v)
        pltpu.sync_copy(part_hbm.at[pl.ds(_PAD + base, _SL_CB)], b_v)

        @plsc.parallel_loop(0, _SL_CB // 16, unroll=8)
        def _add(i):
            a_v[pl.ds(i * 16, 16)] = (a_v[pl.ds(i * 16, 16)] +
                                      b_v[pl.ds(i * 16, 16)])
        pltpu.sync_copy(a_v, out_hbm.at[pl.ds(base, _SL_CB)])

    return segsum_partials, combine


def kernel(indices, per_atom_property):
    segsum_partials, combine = _build()
    idx = indices.astype(jnp.int32)
    val = per_atom_property.reshape(-1).astype(jnp.float32)
    partials = segsum_partials(idx, val)
    full = combine(partials)
    return full[:_N_MOL].reshape(_N_MOL, 1)
